# baseline (speedup 1.0000x reference)
import functools

import jax
import jax.numpy as jnp
from jax import lax
from jax.experimental import pallas as pl
from jax.experimental.pallas import tpu as pltpu
from jax.experimental.pallas import tpu_sc as plsc

B = 16384
D = 64

_info = plsc.get_sparse_core_info()
NC, NS = _info.num_cores, _info.num_subcores
NW = NC * NS
BPW = B // NW
CHUNK = 128
NCHUNK = BPW // CHUNK
NBUF = 6

_mesh = plsc.VectorSubcoreMesh(core_axis_name="c", subcore_axis_name="s")


@functools.partial(
    pl.kernel,
    out_type=jax.ShapeDtypeStruct((B, 2 * D), jnp.float32),
    mesh=_mesh,
    compiler_params=pltpu.CompilerParams(use_tc_tiling_on_sc=False),
    scratch_types=[
        pltpu.VMEM((NCHUNK, CHUNK), jnp.int32),
        pltpu.VMEM((NCHUNK, CHUNK), jnp.int32),
    ]
    + [pltpu.VMEM((CHUNK, 2 * D), jnp.float32) for _ in range(NBUF)]
    + [pltpu.SemaphoreType.DMA for _ in range(NBUF)]
    + [pltpu.SemaphoreType.DMA],
)
def _encode(yt_hbm, w_hbm, out_hbm,
            idx0_v, idx1_v, b0, b1, b2, b3, b4, b5,
            s0, s1, s2, s3, s4, s5, wsem):
    wid = lax.axis_index("s") * NC + lax.axis_index("c")
    base = wid * BPW
    bufs = (b0, b1, b2, b3, b4, b5)
    sems = (s0, s1, s2, s3, s4, s5)

    pltpu.sync_copy(yt_hbm.at[0, pl.ds(wid * NCHUNK, NCHUNK)], idx0_v)
    pltpu.sync_copy(yt_hbm.at[1, pl.ds(wid * NCHUNK, NCHUNK)], idx1_v)

    g0 = [
        pltpu.async_copy(w_hbm.at[idx0_v.at[j]], bufs[j], sems[j])
        for j in range(NCHUNK)
    ]
    g1 = [
        pltpu.async_copy(w_hbm.at[idx1_v.at[j]], bufs[NCHUNK + j],
                         sems[NCHUNK + j])
        for j in range(2)
    ]
    w0 = []
    for j in range(NCHUNK):
        g0[j].wait()
        w0.append(pltpu.async_copy(
            bufs[j], out_hbm.at[pl.ds(base + j * CHUNK, CHUNK)], sems[j]))
    for j in range(2):
        w0[j].wait()
        g1.append(
            pltpu.async_copy(w_hbm.at[idx1_v.at[2 + j]], bufs[j], sems[j]))
    g1_buf = (b4, b5, b0, b1)
    w1 = []
    for j in range(NCHUNK):
        g1[j].wait()
        if j >= 2:
            w0[j].wait()
        w1.append(pltpu.async_copy(
            g1_buf[j].at[:, pl.ds(D, D)],
            out_hbm.at[pl.ds(base + j * CHUNK, CHUNK), pl.ds(D, D)],
            wsem))
    for w in w1:
        w.wait()


def kernel(y, W0, W1):
    yt = y.astype(jnp.int32).T.reshape(2, NW * NCHUNK, CHUNK)
    w = jnp.concatenate([W0, W1], axis=1)
    return _encode(yt, w)


# speedup vs baseline: 1.0011x; 1.0011x over previous
import functools

import jax
import jax.numpy as jnp
from jax import lax
from jax.experimental import pallas as pl
from jax.experimental.pallas import tpu as pltpu
from jax.experimental.pallas import tpu_sc as plsc

B = 16384
D = 64

_info = plsc.get_sparse_core_info()
NC, NS = _info.num_cores, _info.num_subcores
NW = NC * NS
BPW = B // NW
CHUNK = 128
NCHUNK = BPW // CHUNK
NBUF = 6

_mesh = plsc.VectorSubcoreMesh(core_axis_name="c", subcore_axis_name="s")


@functools.partial(
    pl.kernel,
    out_type=jax.ShapeDtypeStruct((B, 2 * D), jnp.float32),
    mesh=_mesh,
    compiler_params=pltpu.CompilerParams(use_tc_tiling_on_sc=False),
    scratch_types=[
        pltpu.VMEM((NCHUNK, CHUNK), jnp.int32),
        pltpu.VMEM((NCHUNK, CHUNK), jnp.int32),
    ]
    + [pltpu.VMEM((CHUNK, 2 * D), jnp.float32) for _ in range(NBUF)]
    + [pltpu.SemaphoreType.DMA for _ in range(NBUF)]
    + [pltpu.SemaphoreType.DMA],
)
def _encode(yt_hbm, w_hbm, out_hbm,
            idx0_v, idx1_v, b0, b1, b2, b3, b4, b5,
            s0, s1, s2, s3, s4, s5, wsem):
    wid = lax.axis_index("s") * NC + lax.axis_index("c")
    base = wid * BPW
    bufs = (b0, b1, b2, b3, b4, b5)
    sems = (s0, s1, s2, s3, s4, s5)

    pltpu.sync_copy(yt_hbm.at[0, pl.ds(wid * NCHUNK, NCHUNK)], idx0_v)
    pltpu.sync_copy(yt_hbm.at[1, pl.ds(wid * NCHUNK, NCHUNK)], idx1_v)

    g0 = [
        pltpu.async_copy(w_hbm.at[idx0_v.at[j]], bufs[j], sems[j])
        for j in range(NCHUNK)
    ]
    g1 = [
        pltpu.async_copy(w_hbm.at[idx1_v.at[j]], bufs[NCHUNK + j],
                         sems[NCHUNK + j])
        for j in range(2)
    ]
    w0 = []
    for j in range(NCHUNK):
        g0[j].wait()
        w0.append(pltpu.async_copy(
            bufs[j], out_hbm.at[pl.ds(base + j * CHUNK, CHUNK)], wsem))
    for w in w0:
        w.wait()
    g1.append(pltpu.async_copy(w_hbm.at[idx1_v.at[2]], b0, s0))
    g1.append(pltpu.async_copy(w_hbm.at[idx1_v.at[3]], b1, s1))
    g1_buf = (b4, b5, b0, b1)
    w1 = []
    for j in range(NCHUNK):
        g1[j].wait()
        w1.append(pltpu.async_copy(
            g1_buf[j].at[:, pl.ds(D, D)],
            out_hbm.at[pl.ds(base + j * CHUNK, CHUNK), pl.ds(D, D)],
            wsem))
    for w in w1:
        w.wait()


def kernel(y, W0, W1):
    yt = y.astype(jnp.int32).T.reshape(2, NW * NCHUNK, CHUNK)
    w = jnp.concatenate([W0, W1], axis=1)
    return _encode(yt, w)
